# baseline (speedup 1.0000x reference)
import functools

import jax
import jax.numpy as jnp
from jax import lax
from jax.experimental import pallas as pl
from jax.experimental.pallas import tpu as pltpu
from jax.experimental.pallas import tpu_sc as plsc

E = 8
K = 2
DIM = 1024
DFF = 4096
EPS = 1e-6

TB = 512
FB = 512


def _routing_body(cap, xf_ref, rmsw_ref, rw_ref, h_ref, meta_ref, cnt_ref):
    i = pl.program_id(0)

    @pl.when(i == 0)
    def _():
        cnt_ref[...] = jnp.zeros_like(cnt_ref)

    x = xf_ref[...]
    ms = jnp.mean(x * x, axis=-1, keepdims=True)
    h = x * lax.rsqrt(ms + EPS) * rmsw_ref[0:1, :]
    h_ref[...] = h

    logits = jnp.dot(h, rw_ref[...], preferred_element_type=jnp.float32)
    z = logits - jnp.max(logits, axis=-1, keepdims=True)
    ez = jnp.exp(z)
    sm = ez / jnp.sum(ez, axis=-1, keepdims=True)

    lanes = lax.broadcasted_iota(jnp.int32, sm.shape, 1)
    m0 = jnp.max(sm, axis=-1, keepdims=True)
    i0 = jnp.min(jnp.where(sm == m0, lanes, E), axis=-1, keepdims=True)
    oneh0 = (lanes == i0).astype(jnp.float32)
    smm = jnp.where(lanes == i0, -1.0, sm)
    m1 = jnp.max(smm, axis=-1, keepdims=True)
    i1 = jnp.min(jnp.where(smm == m1, lanes, E), axis=-1, keepdims=True)
    oneh1 = (lanes == i1).astype(jnp.float32)

    comb = oneh0 + oneh1
    rows = lax.broadcasted_iota(jnp.int32, (TB, TB), 0)
    cols = lax.broadcasted_iota(jnp.int32, (TB, TB), 1)
    tri = (cols < rows).astype(jnp.float32)
    excl = jnp.dot(tri, comb, preferred_element_type=jnp.float32)

    base = cnt_ref[...]
    p0 = jnp.sum((excl + base) * oneh0, axis=-1, keepdims=True)
    p1 = jnp.sum((excl + base + oneh0) * oneh1, axis=-1, keepdims=True)
    cnt_ref[...] = base + jnp.sum(comb, axis=0, keepdims=True)

    capf = jnp.float32(cap)
    k0 = (p0 < capf).astype(jnp.float32)
    k1 = (p1 < capf).astype(jnp.float32)
    p0c = jnp.minimum(p0, capf - 1.0)
    p1c = jnp.minimum(p1, capf - 1.0)

    cols8 = lax.broadcasted_iota(jnp.int32, (TB, E), 1)
    meta = (i0.astype(jnp.float32) * (cols8 == 0)
            + i1.astype(jnp.float32) * (cols8 == 1)
            + p0c * (cols8 == 2)
            + p1c * (cols8 == 3)
            + m0 * (cols8 == 4)
            + m1 * (cols8 == 5)
            + k0 * (cols8 == 6)
            + k1 * (cols8 == 7))
    meta_ref[...] = meta


def _routing(xf, rms_w8, router_w, cap):
    t = xf.shape[0]
    grid = t // TB
    return pl.pallas_call(
        functools.partial(_routing_body, cap),
        grid=(grid,),
        in_specs=[
            pl.BlockSpec((TB, DIM), lambda i: (i, 0)),
            pl.BlockSpec((8, DIM), lambda i: (0, 0)),
            pl.BlockSpec((DIM, E), lambda i: (0, 0)),
        ],
        out_specs=[
            pl.BlockSpec((TB, DIM), lambda i: (i, 0)),
            pl.BlockSpec((TB, E), lambda i: (i, 0)),
        ],
        out_shape=[
            jax.ShapeDtypeStruct((t, DIM), jnp.float32),
            jax.ShapeDtypeStruct((t, E), jnp.float32),
        ],
        scratch_shapes=[pltpu.VMEM((1, E), jnp.float32)],
    )(xf, rms_w8, router_w)


def _make_sc_gather(v, d, b):
    info = plsc.get_sparse_core_info()
    nw = info.num_cores * info.num_subcores
    b_per_w = b // nw
    ch = min(64, b_per_w)
    n_ch = b_per_w // ch
    mesh = plsc.VectorSubcoreMesh(core_axis_name="c", subcore_axis_name="s")

    @functools.partial(
        pl.kernel, mesh=mesh,
        out_type=jax.ShapeDtypeStruct((b, d), jnp.float32),
        scratch_types=[
            pltpu.VMEM((ch,), jnp.int32),
            pltpu.VMEM((ch, d), jnp.float32),
            pltpu.SemaphoreType.DMA,
        ],
    )
    def k(table_hbm, idx_hbm, out_hbm, idx_v, rows_v, sem):
        wid = lax.axis_index("s") * info.num_cores + lax.axis_index("c")
        base = wid * b_per_w
        for c in range(n_ch):
            off = base + c * ch
            pltpu.sync_copy(idx_hbm.at[pl.ds(off, ch)], idx_v)
            pltpu.async_copy(table_hbm.at[idx_v], rows_v, sem).wait()
            pltpu.sync_copy(rows_v, out_hbm.at[pl.ds(off, ch)])

    return k


def _mlp_body(buf_ref, w1_ref, w2_ref, out_ref):
    f = pl.program_id(1)

    @pl.when(f == 0)
    def _():
        out_ref[...] = jnp.zeros_like(out_ref)

    hh = jnp.dot(buf_ref[...].astype(jnp.bfloat16), w1_ref[0],
                 preferred_element_type=jnp.float32)
    hh = jax.nn.gelu(hh, approximate=True)
    out_ref[...] += jnp.dot(hh.astype(jnp.bfloat16), w2_ref[0],
                            preferred_element_type=jnp.float32)


def _mlp(buf, w1, w2, cap):
    return pl.pallas_call(
        _mlp_body,
        grid=(E, DFF // FB),
        in_specs=[
            pl.BlockSpec((cap, DIM), lambda e, f: (e, 0)),
            pl.BlockSpec((1, DIM, FB), lambda e, f: (e, 0, f)),
            pl.BlockSpec((1, FB, DIM), lambda e, f: (e, f, 0)),
        ],
        out_specs=pl.BlockSpec((cap, DIM), lambda e, f: (e, 0)),
        out_shape=jax.ShapeDtypeStruct((E * cap, DIM), jnp.float32),
    )(buf, w1, w2)


def _combine_body(g0_ref, g1_ref, meta_ref, y_ref):
    meta = meta_ref[...]
    a0 = meta[:, 4:5] * meta[:, 6:7]
    a1 = meta[:, 5:6] * meta[:, 7:8]
    y_ref[...] = g0_ref[...] * a0 + g1_ref[...] * a1


def _combine(g, meta, t):
    nb = t // TB
    return pl.pallas_call(
        _combine_body,
        grid=(nb,),
        in_specs=[
            pl.BlockSpec((TB, DIM), lambda i: (i, 0)),
            pl.BlockSpec((TB, DIM), lambda i, _nb=nb: (i + _nb, 0)),
            pl.BlockSpec((TB, E), lambda i: (i, 0)),
        ],
        out_specs=pl.BlockSpec((TB, DIM), lambda i: (i, 0)),
        out_shape=jax.ShapeDtypeStruct((t, DIM), jnp.float32),
    )(g, g, meta)


def kernel(x, rms_w, router_w, w1, w2):
    b, s, _ = x.shape
    t = b * s
    cap = (t * K) // E

    xf = jnp.transpose(x, (1, 0, 2)).reshape(t, DIM)
    rms_w8 = jnp.broadcast_to(rms_w[None, :], (8, DIM))

    h, meta = _routing(xf, rms_w8, router_w, cap)

    e0 = meta[:, 0].astype(jnp.int32)
    e1 = meta[:, 1].astype(jnp.int32)
    p0 = meta[:, 2].astype(jnp.int32)
    p1 = meta[:, 3].astype(jnp.int32)
    k0 = meta[:, 6] > 0.5
    k1 = meta[:, 7] > 0.5
    slot0 = e0 * cap + p0
    slot1 = e1 * cap + p1
    tok = jnp.arange(t, dtype=jnp.int32)
    dump = E * cap
    src = jnp.zeros((E * cap + 8,), jnp.int32)
    src = src.at[jnp.where(k0, slot0, dump)].set(tok)
    src = src.at[jnp.where(k1, slot1, dump)].set(tok)
    src_idx = src[:E * cap]
    comb_idx = jnp.concatenate([jnp.where(k0, slot0, 0),
                                jnp.where(k1, slot1, 0)])

    buf = _make_sc_gather(t, DIM, E * cap)(h, src_idx)
    ob = _mlp(buf, w1.astype(jnp.bfloat16), w2.astype(jnp.bfloat16), cap)
    g = _make_sc_gather(E * cap, DIM, t * K)(ob, comb_idx)
    y = _combine(g, meta, t)

    return jnp.transpose(y.reshape(s, b, DIM), (1, 0, 2))


# speedup vs baseline: 1.2632x; 1.2632x over previous
import functools

import jax
import jax.numpy as jnp
from jax import lax
from jax.experimental import pallas as pl
from jax.experimental.pallas import tpu as pltpu
from jax.experimental.pallas import tpu_sc as plsc

E = 8
K = 2
DIM = 1024
DFF = 4096
EPS = 1e-6

TB = 512
FB = 512


def _routing_body(cap, xf_ref, rmsw_ref, rw_ref, h_ref, meta_ref, cnt_ref):
    i = pl.program_id(0)

    @pl.when(i == 0)
    def _():
        cnt_ref[...] = jnp.zeros_like(cnt_ref)

    x = xf_ref[...]
    ms = jnp.mean(x * x, axis=-1, keepdims=True)
    h = x * lax.rsqrt(ms + EPS) * rmsw_ref[0:1, :]
    h_ref[...] = h

    logits = jnp.dot(h, rw_ref[...], preferred_element_type=jnp.float32)
    z = logits - jnp.max(logits, axis=-1, keepdims=True)
    ez = jnp.exp(z)
    sm = ez / jnp.sum(ez, axis=-1, keepdims=True)

    lanes = lax.broadcasted_iota(jnp.int32, sm.shape, 1)
    m0 = jnp.max(sm, axis=-1, keepdims=True)
    i0 = jnp.min(jnp.where(sm == m0, lanes, E), axis=-1, keepdims=True)
    oneh0 = (lanes == i0).astype(jnp.float32)
    smm = jnp.where(lanes == i0, -1.0, sm)
    m1 = jnp.max(smm, axis=-1, keepdims=True)
    i1 = jnp.min(jnp.where(smm == m1, lanes, E), axis=-1, keepdims=True)
    oneh1 = (lanes == i1).astype(jnp.float32)

    comb = oneh0 + oneh1
    rows = lax.broadcasted_iota(jnp.int32, (TB, TB), 0)
    cols = lax.broadcasted_iota(jnp.int32, (TB, TB), 1)
    tri = (cols < rows).astype(jnp.float32)
    excl = jnp.dot(tri, comb, preferred_element_type=jnp.float32)

    base = cnt_ref[...]
    p0 = jnp.sum((excl + base) * oneh0, axis=-1, keepdims=True)
    p1 = jnp.sum((excl + base + oneh0) * oneh1, axis=-1, keepdims=True)
    cnt_ref[...] = base + jnp.sum(comb, axis=0, keepdims=True)

    capf = jnp.float32(cap)
    k0 = (p0 < capf).astype(jnp.float32)
    k1 = (p1 < capf).astype(jnp.float32)
    p0c = jnp.minimum(p0, capf - 1.0)
    p1c = jnp.minimum(p1, capf - 1.0)

    cols8 = lax.broadcasted_iota(jnp.int32, (TB, E), 1)
    meta = (i0.astype(jnp.float32) * (cols8 == 0)
            + i1.astype(jnp.float32) * (cols8 == 1)
            + p0c * (cols8 == 2)
            + p1c * (cols8 == 3)
            + m0 * (cols8 == 4)
            + m1 * (cols8 == 5)
            + k0 * (cols8 == 6)
            + k1 * (cols8 == 7))
    meta_ref[...] = meta


def _routing(xf, rms_w8, router_w, cap):
    t = xf.shape[0]
    grid = t // TB
    return pl.pallas_call(
        functools.partial(_routing_body, cap),
        grid=(grid,),
        in_specs=[
            pl.BlockSpec((TB, DIM), lambda i: (i, 0)),
            pl.BlockSpec((8, DIM), lambda i: (0, 0)),
            pl.BlockSpec((DIM, E), lambda i: (0, 0)),
        ],
        out_specs=[
            pl.BlockSpec((TB, DIM), lambda i: (i, 0)),
            pl.BlockSpec((TB, E), lambda i: (i, 0)),
        ],
        out_shape=[
            jax.ShapeDtypeStruct((t, DIM), jnp.float32),
            jax.ShapeDtypeStruct((t, E), jnp.float32),
        ],
        scratch_shapes=[pltpu.VMEM((1, E), jnp.float32)],
    )(xf, rms_w8, router_w)


def _make_sc_gather(v, d, b):
    info = plsc.get_sparse_core_info()
    nw = info.num_cores * info.num_subcores
    b_per_w = b // nw
    ch = min(64, b_per_w)
    n_ch = b_per_w // ch
    mesh = plsc.VectorSubcoreMesh(core_axis_name="c", subcore_axis_name="s")

    @functools.partial(
        pl.kernel, mesh=mesh,
        out_type=jax.ShapeDtypeStruct((b, d), jnp.float32),
        scratch_types=[
            pltpu.VMEM((ch,), jnp.int32),
            pltpu.VMEM((ch, d), jnp.float32),
            pltpu.SemaphoreType.DMA,
        ],
    )
    def k(table_hbm, idx_hbm, out_hbm, idx_v, rows_v, sem):
        wid = lax.axis_index("s") * info.num_cores + lax.axis_index("c")
        base = wid * b_per_w
        for c in range(n_ch):
            off = base + c * ch
            pltpu.sync_copy(idx_hbm.at[pl.ds(off, ch)], idx_v)
            pltpu.async_copy(table_hbm.at[idx_v], rows_v, sem).wait()
            pltpu.sync_copy(rows_v, out_hbm.at[pl.ds(off, ch)])

    return k


def _mlp_body(buf_ref, w1_ref, w2_ref, out_ref):
    f = pl.program_id(1)

    @pl.when(f == 0)
    def _():
        out_ref[...] = jnp.zeros_like(out_ref)

    hh = jnp.dot(buf_ref[...].astype(jnp.bfloat16),
                 w1_ref[0].astype(jnp.bfloat16),
                 preferred_element_type=jnp.float32)
    hh = jax.nn.gelu(hh, approximate=True)
    out_ref[...] += jnp.dot(hh.astype(jnp.bfloat16),
                            w2_ref[0].astype(jnp.bfloat16),
                            preferred_element_type=jnp.float32)


def _mlp(buf, w1, w2, cap):
    return pl.pallas_call(
        _mlp_body,
        grid=(E, DFF // FB),
        in_specs=[
            pl.BlockSpec((cap, DIM), lambda e, f: (e, 0)),
            pl.BlockSpec((1, DIM, FB), lambda e, f: (e, 0, f)),
            pl.BlockSpec((1, FB, DIM), lambda e, f: (e, f, 0)),
        ],
        out_specs=pl.BlockSpec((cap, DIM), lambda e, f: (e, 0)),
        out_shape=jax.ShapeDtypeStruct((E * cap, DIM), jnp.float32),
    )(buf, w1, w2)


def _combine_body(g0_ref, g1_ref, meta_ref, y_ref):
    meta = meta_ref[...]
    a0 = meta[:, 4:5] * meta[:, 6:7]
    a1 = meta[:, 5:6] * meta[:, 7:8]
    y_ref[...] = g0_ref[...] * a0 + g1_ref[...] * a1


def _combine(g, meta, t):
    nb = t // TB
    return pl.pallas_call(
        _combine_body,
        grid=(nb,),
        in_specs=[
            pl.BlockSpec((TB, DIM), lambda i: (i, 0)),
            pl.BlockSpec((TB, DIM), lambda i, _nb=nb: (i + _nb, 0)),
            pl.BlockSpec((TB, E), lambda i: (i, 0)),
        ],
        out_specs=pl.BlockSpec((TB, DIM), lambda i: (i, 0)),
        out_shape=jax.ShapeDtypeStruct((t, DIM), jnp.float32),
    )(g, g, meta)


def kernel(x, rms_w, router_w, w1, w2):
    b, s, _ = x.shape
    t = b * s
    cap = (t * K) // E

    xf = jnp.transpose(x, (1, 0, 2)).reshape(t, DIM)
    rms_w8 = jnp.broadcast_to(rms_w[None, :], (8, DIM))

    h, meta = _routing(xf, rms_w8, router_w, cap)

    e0 = meta[:, 0].astype(jnp.int32)
    e1 = meta[:, 1].astype(jnp.int32)
    p0 = meta[:, 2].astype(jnp.int32)
    p1 = meta[:, 3].astype(jnp.int32)
    k0 = meta[:, 6] > 0.5
    k1 = meta[:, 7] > 0.5
    slot0 = e0 * cap + p0
    slot1 = e1 * cap + p1
    tok = jnp.arange(t, dtype=jnp.int32)
    dump = E * cap
    src = jnp.zeros((E * cap + 8,), jnp.int32)
    src = src.at[jnp.where(k0, slot0, dump)].set(tok)
    src = src.at[jnp.where(k1, slot1, dump)].set(tok)
    src_idx = src[:E * cap]
    comb_idx = jnp.concatenate([jnp.where(k0, slot0, 0),
                                jnp.where(k1, slot1, 0)])

    buf = _make_sc_gather(t, DIM, E * cap)(h, src_idx)
    ob = _mlp(buf, w1, w2, cap)
    g = _make_sc_gather(E * cap, DIM, t * K)(ob, comb_idx)
    y = _combine(g, meta, t)

    return jnp.transpose(y.reshape(s, b, DIM), (1, 0, 2))


# speedup vs baseline: 1.3082x; 1.0356x over previous
import functools

import jax
import jax.numpy as jnp
from jax import lax
from jax.experimental import pallas as pl
from jax.experimental.pallas import tpu as pltpu
from jax.experimental.pallas import tpu_sc as plsc

E = 8
K = 2
DIM = 1024
DFF = 4096
EPS = 1e-6

TB = 512
FB = 512


def _routing_body(cap, xf_ref, rmsw_ref, rw_ref, h_ref, meta_ref, cnt_ref):
    i = pl.program_id(0)

    @pl.when(i == 0)
    def _():
        cnt_ref[...] = jnp.zeros_like(cnt_ref)

    x = xf_ref[...]
    ms = jnp.mean(x * x, axis=-1, keepdims=True)
    h = x * lax.rsqrt(ms + EPS) * rmsw_ref[0:1, :]
    h_ref[...] = h

    logits = jnp.dot(h, rw_ref[...], preferred_element_type=jnp.float32)
    z = logits - jnp.max(logits, axis=-1, keepdims=True)
    ez = jnp.exp(z)
    sm = ez / jnp.sum(ez, axis=-1, keepdims=True)

    lanes = lax.broadcasted_iota(jnp.int32, sm.shape, 1)
    m0 = jnp.max(sm, axis=-1, keepdims=True)
    i0 = jnp.min(jnp.where(sm == m0, lanes, E), axis=-1, keepdims=True)
    oneh0 = (lanes == i0).astype(jnp.float32)
    smm = jnp.where(lanes == i0, -1.0, sm)
    m1 = jnp.max(smm, axis=-1, keepdims=True)
    i1 = jnp.min(jnp.where(smm == m1, lanes, E), axis=-1, keepdims=True)
    oneh1 = (lanes == i1).astype(jnp.float32)

    comb = oneh0 + oneh1
    rows = lax.broadcasted_iota(jnp.int32, (TB, TB), 0)
    cols = lax.broadcasted_iota(jnp.int32, (TB, TB), 1)
    tri = (cols < rows).astype(jnp.float32)
    excl = jnp.dot(tri, comb, preferred_element_type=jnp.float32)

    base = cnt_ref[...]
    p0 = jnp.sum((excl + base) * oneh0, axis=-1, keepdims=True)
    p1 = jnp.sum((excl + base + oneh0) * oneh1, axis=-1, keepdims=True)
    cnt_ref[...] = base + jnp.sum(comb, axis=0, keepdims=True)

    capf = jnp.float32(cap)
    k0 = (p0 < capf).astype(jnp.float32)
    k1 = (p1 < capf).astype(jnp.float32)
    p0c = jnp.minimum(p0, capf - 1.0)
    p1c = jnp.minimum(p1, capf - 1.0)

    cols8 = lax.broadcasted_iota(jnp.int32, (TB, E), 1)
    meta = (i0.astype(jnp.float32) * (cols8 == 0)
            + i1.astype(jnp.float32) * (cols8 == 1)
            + p0c * (cols8 == 2)
            + p1c * (cols8 == 3)
            + m0 * (cols8 == 4)
            + m1 * (cols8 == 5)
            + k0 * (cols8 == 6)
            + k1 * (cols8 == 7))
    meta_ref[...] = meta


def _routing(xf, rms_w8, router_w, cap):
    t = xf.shape[0]
    grid = t // TB
    return pl.pallas_call(
        functools.partial(_routing_body, cap),
        grid=(grid,),
        in_specs=[
            pl.BlockSpec((TB, DIM), lambda i: (i, 0)),
            pl.BlockSpec((8, DIM), lambda i: (0, 0)),
            pl.BlockSpec((DIM, E), lambda i: (0, 0)),
        ],
        out_specs=[
            pl.BlockSpec((TB, DIM), lambda i: (i, 0)),
            pl.BlockSpec((TB, E), lambda i: (i, 0)),
        ],
        out_shape=[
            jax.ShapeDtypeStruct((t, DIM), jnp.float32),
            jax.ShapeDtypeStruct((t, E), jnp.float32),
        ],
        scratch_shapes=[pltpu.VMEM((1, E), jnp.float32)],
    )(xf, rms_w8, router_w)


def _make_sc_gather(v, d, b):
    info = plsc.get_sparse_core_info()
    nw = info.num_cores * info.num_subcores
    b_per_w = b // nw
    ch = min(32, b_per_w)
    n_ch = b_per_w // ch
    mesh = plsc.VectorSubcoreMesh(core_axis_name="c", subcore_axis_name="s")

    @functools.partial(
        pl.kernel, mesh=mesh,
        out_type=jax.ShapeDtypeStruct((b, d), jnp.float32),
        scratch_types=[
            pltpu.VMEM((ch,), jnp.int32),
            pltpu.VMEM((ch,), jnp.int32),
            pltpu.VMEM((ch, d), jnp.float32),
            pltpu.VMEM((ch, d), jnp.float32),
            pltpu.SemaphoreType.DMA,
            pltpu.SemaphoreType.DMA,
        ],
    )
    def k(table_hbm, idx_hbm, out_hbm, idx0, idx1, r0, r1, s0, s1):
        wid = lax.axis_index("s") * info.num_cores + lax.axis_index("c")
        base = wid * b_per_w
        idxs, rows, sems = [idx0, idx1], [r0, r1], [s0, s1]
        copies = [None, None]
        pltpu.sync_copy(idx_hbm.at[pl.ds(base, ch)], idx0)
        copies[0] = pltpu.async_copy(table_hbm.at[idx0], r0, s0)
        for c in range(n_ch):
            cur, nxt = c % 2, (c + 1) % 2
            if c + 1 < n_ch:
                off = base + (c + 1) * ch
                pltpu.sync_copy(idx_hbm.at[pl.ds(off, ch)], idxs[nxt])
                copies[nxt] = pltpu.async_copy(
                    table_hbm.at[idxs[nxt]], rows[nxt], sems[nxt])
            copies[cur].wait()
            pltpu.sync_copy(rows[cur], out_hbm.at[pl.ds(base + c * ch, ch)])

    return k


def _mlp_body(buf_ref, w1_ref, w2_ref, out_ref):
    f = pl.program_id(1)

    @pl.when(f == 0)
    def _():
        out_ref[...] = jnp.zeros_like(out_ref)

    hh = jnp.dot(buf_ref[...], w1_ref[0], preferred_element_type=jnp.float32)
    hh = jax.nn.gelu(hh, approximate=True)
    out_ref[...] += jnp.dot(hh, w2_ref[0], preferred_element_type=jnp.float32)


def _mlp(buf, w1, w2, cap):
    return pl.pallas_call(
        _mlp_body,
        grid=(E, DFF // FB),
        in_specs=[
            pl.BlockSpec((cap, DIM), lambda e, f: (e, 0)),
            pl.BlockSpec((1, DIM, FB), lambda e, f: (e, 0, f)),
            pl.BlockSpec((1, FB, DIM), lambda e, f: (e, f, 0)),
        ],
        out_specs=pl.BlockSpec((cap, DIM), lambda e, f: (e, 0)),
        out_shape=jax.ShapeDtypeStruct((E * cap, DIM), jnp.float32),
    )(buf, w1, w2)


def _combine_body(g0_ref, g1_ref, meta_ref, y_ref):
    meta = meta_ref[...]
    a0 = meta[:, 4:5] * meta[:, 6:7]
    a1 = meta[:, 5:6] * meta[:, 7:8]
    y_ref[...] = g0_ref[...] * a0 + g1_ref[...] * a1


def _combine(g, meta, t):
    nb = t // TB
    return pl.pallas_call(
        _combine_body,
        grid=(nb,),
        in_specs=[
            pl.BlockSpec((TB, DIM), lambda i: (i, 0)),
            pl.BlockSpec((TB, DIM), lambda i, _nb=nb: (i + _nb, 0)),
            pl.BlockSpec((TB, E), lambda i: (i, 0)),
        ],
        out_specs=pl.BlockSpec((TB, DIM), lambda i: (i, 0)),
        out_shape=jax.ShapeDtypeStruct((t, DIM), jnp.float32),
    )(g, g, meta)


def kernel(x, rms_w, router_w, w1, w2):
    b, s, _ = x.shape
    t = b * s
    cap = (t * K) // E

    xf = jnp.transpose(x, (1, 0, 2)).reshape(t, DIM)
    rms_w8 = jnp.broadcast_to(rms_w[None, :], (8, DIM))

    h, meta = _routing(xf, rms_w8, router_w, cap)

    e0 = meta[:, 0].astype(jnp.int32)
    e1 = meta[:, 1].astype(jnp.int32)
    p0 = meta[:, 2].astype(jnp.int32)
    p1 = meta[:, 3].astype(jnp.int32)
    k0 = meta[:, 6] > 0.5
    k1 = meta[:, 7] > 0.5
    slot0 = e0 * cap + p0
    slot1 = e1 * cap + p1
    tok = jnp.arange(t, dtype=jnp.int32)
    dump = E * cap
    src = jnp.zeros((E * cap + 8,), jnp.int32)
    src = src.at[jnp.where(k0, slot0, dump)].set(tok)
    src = src.at[jnp.where(k1, slot1, dump)].set(tok)
    src_idx = src[:E * cap]
    comb_idx = jnp.concatenate([jnp.where(k0, slot0, 0),
                                jnp.where(k1, slot1, 0)])

    buf = _make_sc_gather(t, DIM, E * cap)(h, src_idx)
    ob = _mlp(buf, w1, w2, cap)
    g = _make_sc_gather(E * cap, DIM, t * K)(ob, comb_idx)
    y = _combine(g, meta, t)

    return jnp.transpose(y.reshape(s, b, DIM), (1, 0, 2))
